# Initial kernel scaffold; baseline (speedup 1.0000x reference)
import functools

import jax
import jax.numpy as jnp
from jax import lax
from jax.experimental import pallas as pl
from jax.experimental.pallas import tpu as pltpu
from jax.experimental.pallas import tpu_sc as plsc

N = 10000
E = 320000
DIM = 128

NC = 2
NS = 16
NW = NC * NS
EPW = E // NW
CH = 80
NCHUNK = EPW // CH
RPT = N // NS


def _seg_sum_body(src_hbm, dst_hbm, h_hbm, zero_hbm, out_hbm,
                  src_v, dst_v, rows_v, acc_sh, gsem):
    c = lax.axis_index("c")
    s = lax.axis_index("s")
    wid = c * NS + s

    r0 = s * RPT
    pltpu.sync_copy(zero_hbm.at[pl.ds(r0, RPT)], acc_sh.at[pl.ds(r0, RPT)])

    pltpu.sync_copy(src_hbm.at[wid], src_v)
    pltpu.sync_copy(dst_hbm.at[wid], dst_v)
    plsc.subcore_barrier()

    def step(k, carry):
        pltpu.async_copy(h_hbm.at[src_v.at[k]], rows_v, gsem).wait()
        pltpu.sync_copy(rows_v, acc_sh.at[dst_v.at[k]], add=True)
        return carry

    lax.fori_loop(0, NCHUNK, step, 0)
    plsc.subcore_barrier()

    pltpu.sync_copy(acc_sh.at[pl.ds(r0, RPT)],
                    out_hbm.at[pl.ds(c * N + r0, RPT)])


@functools.partial(
    pl.kernel,
    out_type=jax.ShapeDtypeStruct((NC * N, DIM), jnp.float32),
    mesh=plsc.VectorSubcoreMesh(core_axis_name="c", subcore_axis_name="s"),
    scratch_types=[
        pltpu.VMEM((NCHUNK, CH), jnp.int32),
        pltpu.VMEM((NCHUNK, CH), jnp.int32),
        pltpu.VMEM((CH, DIM), jnp.float32),
        pltpu.VMEM_SHARED((N, DIM), jnp.float32),
        pltpu.SemaphoreType.DMA,
    ],
)
def _segment_sum_sc(src_hbm, dst_hbm, h_hbm, zero_hbm, out_hbm,
                    src_v, dst_v, rows_v, acc_sh, gsem):
    _seg_sum_body(src_hbm, dst_hbm, h_hbm, zero_hbm, out_hbm,
                  src_v, dst_v, rows_v, acc_sh, gsem)


BR = 1000


def _embed_body(x_ref, w_ref, o_ref):
    o_ref[...] = jnp.dot(x_ref[...], w_ref[...],
                         preferred_element_type=jnp.float32)


def _embed(x, w):
    return pl.pallas_call(
        _embed_body,
        grid=(N // BR,),
        in_specs=[
            pl.BlockSpec((BR, DIM), lambda i: (i, 0)),
            pl.BlockSpec((DIM, DIM), lambda i: (0, 0)),
        ],
        out_specs=pl.BlockSpec((BR, DIM), lambda i: (i, 0)),
        out_shape=jax.ShapeDtypeStruct((N, DIM), jnp.float32),
    )(x, w)


def _mlp_body(relu_out, eps_ref, h_ref, agg_ref, wa_ref, ba_ref, wb_ref,
              bb_ref, o_ref):
    eps = eps_ref[0, 0]
    z = (1.0 + eps) * h_ref[...] + agg_ref[0] + agg_ref[1]
    a = jnp.dot(z, wa_ref[...], preferred_element_type=jnp.float32)
    a = jnp.maximum(a + ba_ref[...], 0.0)
    y = jnp.dot(a, wb_ref[...], preferred_element_type=jnp.float32)
    y = y + bb_ref[...]
    if relu_out:
        y = jnp.maximum(y, 0.0)
    o_ref[...] = y


def _mlp(relu_out, eps, h, agg, wa, ba, wb, bb):
    hid = wa.shape[1]
    return pl.pallas_call(
        functools.partial(_mlp_body, relu_out),
        grid=(N // BR,),
        in_specs=[
            pl.BlockSpec(memory_space=pltpu.SMEM),
            pl.BlockSpec((BR, DIM), lambda i: (i, 0)),
            pl.BlockSpec((NC, BR, DIM), lambda i: (0, i, 0)),
            pl.BlockSpec((DIM, hid), lambda i: (0, 0)),
            pl.BlockSpec((1, hid), lambda i: (0, 0)),
            pl.BlockSpec((hid, DIM), lambda i: (0, 0)),
            pl.BlockSpec((1, DIM), lambda i: (0, 0)),
        ],
        out_specs=pl.BlockSpec((BR, DIM), lambda i: (i, 0)),
        out_shape=jax.ShapeDtypeStruct((N, DIM), jnp.float32),
    )(eps, h, agg, wa, ba, wb, bb)


def kernel(x, edge_index, W_emb,
           eps1, W1a, b1a, W1b, b1b,
           eps2, W2a, b2a, W2b, b2b,
           eps3, W3a, b3a, W3b, b3b):
    src3 = edge_index[0].reshape(NW, NCHUNK, CH)
    dst3 = edge_index[1].reshape(NW, NCHUNK, CH)
    zeros = jnp.zeros((N, DIM), jnp.float32)

    h = _embed(x, W_emb)
    layers = [
        (eps1, W1a, b1a, W1b, b1b, False),
        (eps2, W2a, b2a, W2b, b2b, True),
        (eps3, W3a, b3a, W3b, b3b, True),
    ]
    for eps, wa, ba, wb, bb, relu_out in layers:
        part = _segment_sum_sc(src3, dst3, h, zeros)
        agg = part.reshape(NC, N, DIM)
        h = _mlp(relu_out, jnp.reshape(eps, (1, 1)), h, agg,
                 wa, jnp.reshape(ba, (1, -1)), wb, jnp.reshape(bb, (1, -1)))
    return h


# speedup vs baseline: 6.4407x; 6.4407x over previous
import functools

import jax
import jax.numpy as jnp
from jax import lax
from jax.experimental import pallas as pl
from jax.experimental.pallas import tpu as pltpu
from jax.experimental.pallas import tpu_sc as plsc

N = 10000
E = 320000
DIM = 128

NC = 2
NS = 16
NW = NC * NS
EPW = E // NW
CH = 80
NCHUNK = EPW // CH
RPT = 624
TAIL0 = NS * RPT
TAILN = N - TAIL0


def _seg_sum_body(src_hbm, dst_hbm, h_hbm, zero_hbm, out_hbm,
                  src_v, dst_v, rows_v, acc_sh, gsem):
    c = lax.axis_index("c")
    s = lax.axis_index("s")
    wid = c * NS + s

    r0 = pl.multiple_of(s * RPT, 8)
    pltpu.sync_copy(zero_hbm.at[pl.ds(r0, RPT)], acc_sh.at[pl.ds(r0, RPT)])

    @pl.when(s == NS - 1)
    def _zero_tail():
        pltpu.sync_copy(zero_hbm.at[pl.ds(TAIL0, TAILN)],
                        acc_sh.at[pl.ds(TAIL0, TAILN)])

    pltpu.sync_copy(src_hbm.at[wid], src_v)
    pltpu.sync_copy(dst_hbm.at[wid], dst_v)
    plsc.subcore_barrier()

    def step(k, carry):
        pltpu.async_copy(h_hbm.at[src_v.at[k]], rows_v, gsem).wait()
        pltpu.sync_copy(rows_v, acc_sh.at[dst_v.at[k]], add=True)
        return carry

    lax.fori_loop(0, NCHUNK, step, 0)
    plsc.subcore_barrier()

    o0 = pl.multiple_of(c * N + r0, 8)
    pltpu.sync_copy(acc_sh.at[pl.ds(r0, RPT)], out_hbm.at[pl.ds(o0, RPT)])

    @pl.when(s == NS - 1)
    def _out_tail():
        ot = pl.multiple_of(c * N + TAIL0, 8)
        pltpu.sync_copy(acc_sh.at[pl.ds(TAIL0, TAILN)],
                        out_hbm.at[pl.ds(ot, TAILN)])


@functools.partial(
    pl.kernel,
    out_type=jax.ShapeDtypeStruct((NC * N, DIM), jnp.float32),
    mesh=plsc.VectorSubcoreMesh(core_axis_name="c", subcore_axis_name="s"),
    scratch_types=[
        pltpu.VMEM((NCHUNK, CH), jnp.int32),
        pltpu.VMEM((NCHUNK, CH), jnp.int32),
        pltpu.VMEM((CH, DIM), jnp.float32),
        pltpu.VMEM_SHARED((N, DIM), jnp.float32),
        pltpu.SemaphoreType.DMA,
    ],
)
def _segment_sum_sc(src_hbm, dst_hbm, h_hbm, zero_hbm, out_hbm,
                    src_v, dst_v, rows_v, acc_sh, gsem):
    _seg_sum_body(src_hbm, dst_hbm, h_hbm, zero_hbm, out_hbm,
                  src_v, dst_v, rows_v, acc_sh, gsem)


BR = 1000


def _embed_body(x_ref, w_ref, o_ref):
    o_ref[...] = jnp.dot(x_ref[...], w_ref[...],
                         preferred_element_type=jnp.float32)


def _embed(x, w):
    return pl.pallas_call(
        _embed_body,
        grid=(N // BR,),
        in_specs=[
            pl.BlockSpec((BR, DIM), lambda i: (i, 0)),
            pl.BlockSpec((DIM, DIM), lambda i: (0, 0)),
        ],
        out_specs=pl.BlockSpec((BR, DIM), lambda i: (i, 0)),
        out_shape=jax.ShapeDtypeStruct((N, DIM), jnp.float32),
    )(x, w)


def _mlp_body(relu_out, eps_ref, h_ref, agg_ref, wa_ref, ba_ref, wb_ref,
              bb_ref, o_ref):
    eps = eps_ref[0, 0]
    z = (1.0 + eps) * h_ref[...] + agg_ref[0] + agg_ref[1]
    a = jnp.dot(z, wa_ref[...], preferred_element_type=jnp.float32)
    a = jnp.maximum(a + ba_ref[...], 0.0)
    y = jnp.dot(a, wb_ref[...], preferred_element_type=jnp.float32)
    y = y + bb_ref[...]
    if relu_out:
        y = jnp.maximum(y, 0.0)
    o_ref[...] = y


def _mlp(relu_out, eps, h, agg, wa, ba, wb, bb):
    hid = wa.shape[1]
    return pl.pallas_call(
        functools.partial(_mlp_body, relu_out),
        grid=(N // BR,),
        in_specs=[
            pl.BlockSpec(memory_space=pltpu.SMEM),
            pl.BlockSpec((BR, DIM), lambda i: (i, 0)),
            pl.BlockSpec((NC, BR, DIM), lambda i: (0, i, 0)),
            pl.BlockSpec((DIM, hid), lambda i: (0, 0)),
            pl.BlockSpec((1, hid), lambda i: (0, 0)),
            pl.BlockSpec((hid, DIM), lambda i: (0, 0)),
            pl.BlockSpec((1, DIM), lambda i: (0, 0)),
        ],
        out_specs=pl.BlockSpec((BR, DIM), lambda i: (i, 0)),
        out_shape=jax.ShapeDtypeStruct((N, DIM), jnp.float32),
    )(eps, h, agg, wa, ba, wb, bb)


def kernel(x, edge_index, W_emb,
           eps1, W1a, b1a, W1b, b1b,
           eps2, W2a, b2a, W2b, b2b,
           eps3, W3a, b3a, W3b, b3b):
    src3 = edge_index[0].reshape(NW, NCHUNK, CH)
    dst3 = edge_index[1].reshape(NW, NCHUNK, CH)
    zeros = jnp.zeros((N, DIM), jnp.float32)

    h = _embed(x, W_emb)
    layers = [
        (eps1, W1a, b1a, W1b, b1b, False),
        (eps2, W2a, b2a, W2b, b2b, True),
        (eps3, W3a, b3a, W3b, b3b, True),
    ]
    for eps, wa, ba, wb, bb, relu_out in layers:
        part = _segment_sum_sc(src3, dst3, h, zeros)
        agg = part.reshape(NC, N, DIM)
        h = _mlp(relu_out, jnp.reshape(eps, (1, 1)), h, agg,
                 wa, jnp.reshape(ba, (1, -1)), wb, jnp.reshape(bb, (1, -1)))
    return h


# speedup vs baseline: 10.1535x; 1.5765x over previous
import functools

import jax
import jax.numpy as jnp
from jax import lax
from jax.experimental import pallas as pl
from jax.experimental.pallas import tpu as pltpu
from jax.experimental.pallas import tpu_sc as plsc

N = 10000
E = 320000
DIM = 128

NC = 2
NS = 16
NW = NC * NS
EPW = E // NW
CH = 80
NCHUNK = EPW // CH
NB = 2
GC = 25
NG = NCHUNK // GC
RPT = 624
TAIL0 = NS * RPT
TAILN = N - TAIL0


def _seg_sum_body(src_hbm, dst_hbm, h_hbm, zero_hbm, out_hbm,
                  src_v, dst_v, rows_v, acc_sh, gsem, ssem, isem):
    c = lax.axis_index("c")
    s = lax.axis_index("s")
    wid = c * NS + s

    r0 = pl.multiple_of(s * RPT, 8)
    pltpu.sync_copy(zero_hbm.at[pl.ds(r0, RPT)], acc_sh.at[pl.ds(r0, RPT)])

    @pl.when(s == NS - 1)
    def _zero_tail():
        pltpu.sync_copy(zero_hbm.at[pl.ds(TAIL0, TAILN)],
                        acc_sh.at[pl.ds(TAIL0, TAILN)])

    def _idx_start(g, gb):
        pltpu.make_async_copy(src_hbm.at[wid * NG + g],
                              src_v.at[pl.ds(gb * GC, GC)], isem).start()
        pltpu.make_async_copy(dst_hbm.at[wid * NG + g],
                              dst_v.at[pl.ds(gb * GC, GC)], isem).start()

    def _idx_wait(g, gb):
        pltpu.make_async_copy(src_hbm.at[wid * NG + g],
                              src_v.at[pl.ds(gb * GC, GC)], isem).wait()
        pltpu.make_async_copy(dst_hbm.at[wid * NG + g],
                              dst_v.at[pl.ds(gb * GC, GC)], isem).wait()

    _idx_start(0, 0)
    plsc.subcore_barrier()

    def _gather(r, b):
        return pltpu.make_async_copy(h_hbm.at[src_v.at[r]], rows_v.at[b], gsem)

    def _scatter(r, b):
        return pltpu.async_copy(rows_v.at[b], acc_sh.at[dst_v.at[r]], ssem,
                                add=True)

    def _scatter_wait(r, b):
        pltpu.make_async_copy(rows_v.at[b], acc_sh.at[dst_v.at[r]],
                              ssem).wait()

    def group(g, carry):
        gb = lax.rem(g, 2)
        _idx_wait(g, gb)

        @pl.when(g + 1 < NG)
        def _prefetch_idx():
            _idx_start(g + 1, 1 - gb)

        row0 = gb * GC
        _gather(row0, 0).start()

        def step(j, carry):
            b = lax.rem(j, NB)

            @pl.when(j >= 1)
            def _drain_scatter():
                _scatter_wait(row0 + j - 1, 1 - b)

            @pl.when(j + 1 < GC)
            def _issue_gather():
                _gather(row0 + j + 1, 1 - b).start()

            _gather(row0 + j, b).wait()
            _scatter(row0 + j, b)
            return carry

        lax.fori_loop(0, GC, step, 0)
        _scatter_wait(row0 + GC - 1, (GC - 1) % NB)
        return carry

    lax.fori_loop(0, NG, group, 0)
    plsc.subcore_barrier()

    o0 = pl.multiple_of(c * N + r0, 8)
    pltpu.sync_copy(acc_sh.at[pl.ds(r0, RPT)], out_hbm.at[pl.ds(o0, RPT)])

    @pl.when(s == NS - 1)
    def _out_tail():
        ot = pl.multiple_of(c * N + TAIL0, 8)
        pltpu.sync_copy(acc_sh.at[pl.ds(TAIL0, TAILN)],
                        out_hbm.at[pl.ds(ot, TAILN)])


@functools.partial(
    pl.kernel,
    out_type=jax.ShapeDtypeStruct((NC * N, DIM), jnp.float32),
    mesh=plsc.VectorSubcoreMesh(core_axis_name="c", subcore_axis_name="s"),
    scratch_types=[
        pltpu.VMEM((2 * GC, CH), jnp.int32),
        pltpu.VMEM((2 * GC, CH), jnp.int32),
        pltpu.VMEM((NB, CH, DIM), jnp.float32),
        pltpu.VMEM_SHARED((N, DIM), jnp.float32),
        pltpu.SemaphoreType.DMA,
        pltpu.SemaphoreType.DMA,
        pltpu.SemaphoreType.DMA,
    ],
)
def _segment_sum_sc(src_hbm, dst_hbm, h_hbm, zero_hbm, out_hbm,
                    src_v, dst_v, rows_v, acc_sh, gsem, ssem, isem):
    _seg_sum_body(src_hbm, dst_hbm, h_hbm, zero_hbm, out_hbm,
                  src_v, dst_v, rows_v, acc_sh, gsem, ssem, isem)


BR = 1000


def _embed_body(x_ref, w_ref, o_ref):
    o_ref[...] = jnp.dot(x_ref[...], w_ref[...],
                         preferred_element_type=jnp.float32)


def _embed(x, w):
    return pl.pallas_call(
        _embed_body,
        grid=(N // BR,),
        in_specs=[
            pl.BlockSpec((BR, DIM), lambda i: (i, 0)),
            pl.BlockSpec((DIM, DIM), lambda i: (0, 0)),
        ],
        out_specs=pl.BlockSpec((BR, DIM), lambda i: (i, 0)),
        out_shape=jax.ShapeDtypeStruct((N, DIM), jnp.float32),
    )(x, w)


def _mlp_body(relu_out, eps_ref, h_ref, agg_ref, wa_ref, ba_ref, wb_ref,
              bb_ref, o_ref):
    eps = eps_ref[0, 0]
    z = (1.0 + eps) * h_ref[...] + agg_ref[0] + agg_ref[1]
    a = jnp.dot(z, wa_ref[...], preferred_element_type=jnp.float32)
    a = jnp.maximum(a + ba_ref[...], 0.0)
    y = jnp.dot(a, wb_ref[...], preferred_element_type=jnp.float32)
    y = y + bb_ref[...]
    if relu_out:
        y = jnp.maximum(y, 0.0)
    o_ref[...] = y


def _mlp(relu_out, eps, h, agg, wa, ba, wb, bb):
    hid = wa.shape[1]
    return pl.pallas_call(
        functools.partial(_mlp_body, relu_out),
        grid=(N // BR,),
        in_specs=[
            pl.BlockSpec(memory_space=pltpu.SMEM),
            pl.BlockSpec((BR, DIM), lambda i: (i, 0)),
            pl.BlockSpec((NC, BR, DIM), lambda i: (0, i, 0)),
            pl.BlockSpec((DIM, hid), lambda i: (0, 0)),
            pl.BlockSpec((1, hid), lambda i: (0, 0)),
            pl.BlockSpec((hid, DIM), lambda i: (0, 0)),
            pl.BlockSpec((1, DIM), lambda i: (0, 0)),
        ],
        out_specs=pl.BlockSpec((BR, DIM), lambda i: (i, 0)),
        out_shape=jax.ShapeDtypeStruct((N, DIM), jnp.float32),
    )(eps, h, agg, wa, ba, wb, bb)


def kernel(x, edge_index, W_emb,
           eps1, W1a, b1a, W1b, b1b,
           eps2, W2a, b2a, W2b, b2b,
           eps3, W3a, b3a, W3b, b3b):
    src3 = edge_index[0].reshape(NW * NG, GC, CH)
    dst3 = edge_index[1].reshape(NW * NG, GC, CH)
    zeros = jnp.zeros((N, DIM), jnp.float32)

    h = _embed(x, W_emb)
    layers = [
        (eps1, W1a, b1a, W1b, b1b, False),
        (eps2, W2a, b2a, W2b, b2b, True),
        (eps3, W3a, b3a, W3b, b3b, True),
    ]
    for eps, wa, ba, wb, bb, relu_out in layers:
        part = _segment_sum_sc(src3, dst3, h, zeros)
        agg = part.reshape(NC, N, DIM)
        h = _mlp(relu_out, jnp.reshape(eps, (1, 1)), h, agg,
                 wa, jnp.reshape(ba, (1, -1)), wb, jnp.reshape(bb, (1, -1)))
    return h


# speedup vs baseline: 11.8093x; 1.1631x over previous
import functools

import jax
import jax.numpy as jnp
from jax import lax
from jax.experimental import pallas as pl
from jax.experimental.pallas import tpu as pltpu
from jax.experimental.pallas import tpu_sc as plsc

N = 10000
E = 320000
DIM = 128

NC = 2
NS = 16
NW = NC * NS
EPW = E // NW
CH = 80
NCHUNK = EPW // CH
NB = 3
GC = 25
NG = NCHUNK // GC
RPT = 624
TAIL0 = NS * RPT
TAILN = N - TAIL0


def _seg_sum_body(src_hbm, dst_hbm, h_hbm, zero_hbm, out_hbm,
                  src_v, dst_v, rows_v, acc_sh, gsem, ssem, isem):
    c = lax.axis_index("c")
    s = lax.axis_index("s")
    wid = c * NS + s

    r0 = pl.multiple_of(s * RPT, 8)
    pltpu.sync_copy(zero_hbm.at[pl.ds(r0, RPT)], acc_sh.at[pl.ds(r0, RPT)])

    @pl.when(s == NS - 1)
    def _zero_tail():
        pltpu.sync_copy(zero_hbm.at[pl.ds(TAIL0, TAILN)],
                        acc_sh.at[pl.ds(TAIL0, TAILN)])

    def _idx_start(g, gb):
        pltpu.make_async_copy(src_hbm.at[wid * NG + g],
                              src_v.at[pl.ds(gb * GC, GC)], isem).start()
        pltpu.make_async_copy(dst_hbm.at[wid * NG + g],
                              dst_v.at[pl.ds(gb * GC, GC)], isem).start()

    def _idx_wait(g, gb):
        pltpu.make_async_copy(src_hbm.at[wid * NG + g],
                              src_v.at[pl.ds(gb * GC, GC)], isem).wait()
        pltpu.make_async_copy(dst_hbm.at[wid * NG + g],
                              dst_v.at[pl.ds(gb * GC, GC)], isem).wait()

    _idx_start(0, 0)
    plsc.subcore_barrier()

    def _gather(j, b):
        r = lax.rem(j, 2 * GC)
        return pltpu.make_async_copy(h_hbm.at[src_v.at[r]], rows_v.at[b], gsem)

    def _scatter(j, b):
        r = lax.rem(j, 2 * GC)
        return pltpu.async_copy(rows_v.at[b], acc_sh.at[dst_v.at[r]], ssem,
                                add=True)

    def _scatter_wait(j, b):
        r = lax.rem(j, 2 * GC)
        pltpu.make_async_copy(rows_v.at[b], acc_sh.at[dst_v.at[r]],
                              ssem).wait()

    _idx_wait(0, 0)

    @pl.when(NG > 1)
    def _prefetch_first():
        _idx_start(1, 1)

    _gather(0, 0).start()

    def step(j, carry):
        b = lax.rem(j, NB)

        @pl.when(j >= 2)
        def _drain_scatter():
            _scatter_wait(j - 2, lax.rem(j + 1, NB))

        @pl.when(j + 1 < NCHUNK)
        def _issue_gather():
            nj = j + 1
            g = nj // GC
            gb = lax.rem(g, 2)

            @pl.when(lax.rem(nj, GC) == 0)
            def _rotate_idx_group():
                _idx_wait(g, gb)

            @pl.when((lax.rem(nj, GC) == 3) & (nj >= GC) & (g + 1 < NG))
            def _prefetch_next():
                _idx_start(g + 1, 1 - gb)

            _gather(nj, lax.rem(nj, NB)).start()

        _gather(j, b).wait()
        _scatter(j, b)
        return carry

    lax.fori_loop(0, NCHUNK, step, 0)
    _scatter_wait(NCHUNK - 2, lax.rem(NCHUNK - 2, NB))
    _scatter_wait(NCHUNK - 1, lax.rem(NCHUNK - 1, NB))
    plsc.subcore_barrier()

    o0 = pl.multiple_of(c * N + r0, 8)
    pltpu.sync_copy(acc_sh.at[pl.ds(r0, RPT)], out_hbm.at[pl.ds(o0, RPT)])

    @pl.when(s == NS - 1)
    def _out_tail():
        ot = pl.multiple_of(c * N + TAIL0, 8)
        pltpu.sync_copy(acc_sh.at[pl.ds(TAIL0, TAILN)],
                        out_hbm.at[pl.ds(ot, TAILN)])


@functools.partial(
    pl.kernel,
    out_type=jax.ShapeDtypeStruct((NC * N, DIM), jnp.float32),
    mesh=plsc.VectorSubcoreMesh(core_axis_name="c", subcore_axis_name="s"),
    scratch_types=[
        pltpu.VMEM((2 * GC, CH), jnp.int32),
        pltpu.VMEM((2 * GC, CH), jnp.int32),
        pltpu.VMEM((NB, CH, DIM), jnp.float32),
        pltpu.VMEM_SHARED((N, DIM), jnp.float32),
        pltpu.SemaphoreType.DMA,
        pltpu.SemaphoreType.DMA,
        pltpu.SemaphoreType.DMA,
    ],
)
def _segment_sum_sc(src_hbm, dst_hbm, h_hbm, zero_hbm, out_hbm,
                    src_v, dst_v, rows_v, acc_sh, gsem, ssem, isem):
    _seg_sum_body(src_hbm, dst_hbm, h_hbm, zero_hbm, out_hbm,
                  src_v, dst_v, rows_v, acc_sh, gsem, ssem, isem)


BR = 1000


def _embed_body(x_ref, w_ref, o_ref):
    o_ref[...] = jnp.dot(x_ref[...], w_ref[...],
                         preferred_element_type=jnp.float32)


def _embed(x, w):
    return pl.pallas_call(
        _embed_body,
        grid=(N // BR,),
        in_specs=[
            pl.BlockSpec((BR, DIM), lambda i: (i, 0)),
            pl.BlockSpec((DIM, DIM), lambda i: (0, 0)),
        ],
        out_specs=pl.BlockSpec((BR, DIM), lambda i: (i, 0)),
        out_shape=jax.ShapeDtypeStruct((N, DIM), jnp.float32),
    )(x, w)


def _mlp_body(relu_out, eps_ref, h_ref, agg_ref, wa_ref, ba_ref, wb_ref,
              bb_ref, o_ref):
    eps = eps_ref[0, 0]
    z = (1.0 + eps) * h_ref[...] + agg_ref[0] + agg_ref[1]
    a = jnp.dot(z, wa_ref[...], preferred_element_type=jnp.float32)
    a = jnp.maximum(a + ba_ref[...], 0.0)
    y = jnp.dot(a, wb_ref[...], preferred_element_type=jnp.float32)
    y = y + bb_ref[...]
    if relu_out:
        y = jnp.maximum(y, 0.0)
    o_ref[...] = y


def _mlp(relu_out, eps, h, agg, wa, ba, wb, bb):
    hid = wa.shape[1]
    return pl.pallas_call(
        functools.partial(_mlp_body, relu_out),
        grid=(N // BR,),
        in_specs=[
            pl.BlockSpec(memory_space=pltpu.SMEM),
            pl.BlockSpec((BR, DIM), lambda i: (i, 0)),
            pl.BlockSpec((NC, BR, DIM), lambda i: (0, i, 0)),
            pl.BlockSpec((DIM, hid), lambda i: (0, 0)),
            pl.BlockSpec((1, hid), lambda i: (0, 0)),
            pl.BlockSpec((hid, DIM), lambda i: (0, 0)),
            pl.BlockSpec((1, DIM), lambda i: (0, 0)),
        ],
        out_specs=pl.BlockSpec((BR, DIM), lambda i: (i, 0)),
        out_shape=jax.ShapeDtypeStruct((N, DIM), jnp.float32),
    )(eps, h, agg, wa, ba, wb, bb)


def kernel(x, edge_index, W_emb,
           eps1, W1a, b1a, W1b, b1b,
           eps2, W2a, b2a, W2b, b2b,
           eps3, W3a, b3a, W3b, b3b):
    src3 = edge_index[0].reshape(NW * NG, GC, CH)
    dst3 = edge_index[1].reshape(NW * NG, GC, CH)
    zeros = jnp.zeros((N, DIM), jnp.float32)

    h = _embed(x, W_emb)
    layers = [
        (eps1, W1a, b1a, W1b, b1b, False),
        (eps2, W2a, b2a, W2b, b2b, True),
        (eps3, W3a, b3a, W3b, b3b, True),
    ]
    for eps, wa, ba, wb, bb, relu_out in layers:
        part = _segment_sum_sc(src3, dst3, h, zeros)
        agg = part.reshape(NC, N, DIM)
        h = _mlp(relu_out, jnp.reshape(eps, (1, 1)), h, agg,
                 wa, jnp.reshape(ba, (1, -1)), wb, jnp.reshape(bb, (1, -1)))
    return h


# speedup vs baseline: 12.2088x; 1.0338x over previous
import functools

import jax
import jax.numpy as jnp
from jax import lax
from jax.experimental import pallas as pl
from jax.experimental.pallas import tpu as pltpu
from jax.experimental.pallas import tpu_sc as plsc

N = 10000
E = 320000
DIM = 128

NC = 2
NS = 16
NW = NC * NS
EPW = E // NW
CH = 100
NCHUNK = EPW // CH
NB = 3
GC = 5
NG = NCHUNK // GC
RPT = 624
TAIL0 = NS * RPT
TAILN = N - TAIL0


def _seg_sum_body(src_hbm, dst_hbm, h_hbm, zero_hbm, out_hbm,
                  src_v, dst_v, rows_v, acc_sh, gsem, ssem, isem):
    c = lax.axis_index("c")
    s = lax.axis_index("s")
    wid = c * NS + s

    r0 = pl.multiple_of(s * RPT, 8)
    pltpu.sync_copy(zero_hbm.at[pl.ds(r0, RPT)], acc_sh.at[pl.ds(r0, RPT)])

    @pl.when(s == NS - 1)
    def _zero_tail():
        pltpu.sync_copy(zero_hbm.at[pl.ds(TAIL0, TAILN)],
                        acc_sh.at[pl.ds(TAIL0, TAILN)])

    def _idx_start(g, gb):
        pltpu.make_async_copy(src_hbm.at[wid * NG + g],
                              src_v.at[pl.ds(gb * GC, GC)], isem).start()
        pltpu.make_async_copy(dst_hbm.at[wid * NG + g],
                              dst_v.at[pl.ds(gb * GC, GC)], isem).start()

    def _idx_wait(g, gb):
        pltpu.make_async_copy(src_hbm.at[wid * NG + g],
                              src_v.at[pl.ds(gb * GC, GC)], isem).wait()
        pltpu.make_async_copy(dst_hbm.at[wid * NG + g],
                              dst_v.at[pl.ds(gb * GC, GC)], isem).wait()

    _idx_start(0, 0)
    plsc.subcore_barrier()

    def _gather(j, b):
        r = lax.rem(j, 2 * GC)
        return pltpu.make_async_copy(h_hbm.at[src_v.at[r]], rows_v.at[b], gsem)

    def _scatter(j, b):
        r = lax.rem(j, 2 * GC)
        return pltpu.async_copy(rows_v.at[b], acc_sh.at[dst_v.at[r]], ssem,
                                add=True)

    def _scatter_wait(j, b):
        r = lax.rem(j, 2 * GC)
        pltpu.make_async_copy(rows_v.at[b], acc_sh.at[dst_v.at[r]],
                              ssem).wait()

    _idx_wait(0, 0)

    @pl.when(NG > 1)
    def _prefetch_first():
        _idx_start(1, 1)

    _gather(0, 0).start()

    def step(j, carry):
        b = lax.rem(j, NB)

        @pl.when(j >= 2)
        def _drain_scatter():
            _scatter_wait(j - 2, lax.rem(j + 1, NB))

        @pl.when(j + 1 < NCHUNK)
        def _issue_gather():
            nj = j + 1
            g = nj // GC
            gb = lax.rem(g, 2)

            @pl.when(lax.rem(nj, GC) == 0)
            def _rotate_idx_group():
                _idx_wait(g, gb)

            @pl.when((lax.rem(nj, GC) == 3) & (nj >= GC) & (g + 1 < NG))
            def _prefetch_next():
                _idx_start(g + 1, 1 - gb)

            _gather(nj, lax.rem(nj, NB)).start()

        _gather(j, b).wait()
        _scatter(j, b)
        return carry

    lax.fori_loop(0, NCHUNK, step, 0)
    _scatter_wait(NCHUNK - 2, lax.rem(NCHUNK - 2, NB))
    _scatter_wait(NCHUNK - 1, lax.rem(NCHUNK - 1, NB))
    plsc.subcore_barrier()

    o0 = pl.multiple_of(c * N + r0, 8)
    pltpu.sync_copy(acc_sh.at[pl.ds(r0, RPT)], out_hbm.at[pl.ds(o0, RPT)])

    @pl.when(s == NS - 1)
    def _out_tail():
        ot = pl.multiple_of(c * N + TAIL0, 8)
        pltpu.sync_copy(acc_sh.at[pl.ds(TAIL0, TAILN)],
                        out_hbm.at[pl.ds(ot, TAILN)])


@functools.partial(
    pl.kernel,
    out_type=jax.ShapeDtypeStruct((NC * N, DIM), jnp.float32),
    mesh=plsc.VectorSubcoreMesh(core_axis_name="c", subcore_axis_name="s"),
    scratch_types=[
        pltpu.VMEM((2 * GC, CH), jnp.int32),
        pltpu.VMEM((2 * GC, CH), jnp.int32),
        pltpu.VMEM((NB, CH, DIM), jnp.float32),
        pltpu.VMEM_SHARED((N, DIM), jnp.float32),
        pltpu.SemaphoreType.DMA,
        pltpu.SemaphoreType.DMA,
        pltpu.SemaphoreType.DMA,
    ],
)
def _segment_sum_sc(src_hbm, dst_hbm, h_hbm, zero_hbm, out_hbm,
                    src_v, dst_v, rows_v, acc_sh, gsem, ssem, isem):
    _seg_sum_body(src_hbm, dst_hbm, h_hbm, zero_hbm, out_hbm,
                  src_v, dst_v, rows_v, acc_sh, gsem, ssem, isem)


BR = 1000


def _embed_body(x_ref, w_ref, o_ref):
    o_ref[...] = jnp.dot(x_ref[...].astype(jnp.bfloat16),
                         w_ref[...].astype(jnp.bfloat16),
                         preferred_element_type=jnp.float32)


def _embed(x, w):
    return pl.pallas_call(
        _embed_body,
        grid=(N // BR,),
        in_specs=[
            pl.BlockSpec((BR, DIM), lambda i: (i, 0)),
            pl.BlockSpec((DIM, DIM), lambda i: (0, 0)),
        ],
        out_specs=pl.BlockSpec((BR, DIM), lambda i: (i, 0)),
        out_shape=jax.ShapeDtypeStruct((N, DIM), jnp.float32),
    )(x, w)


def _mlp_body(relu_out, eps_ref, h_ref, agg_ref, wa_ref, ba_ref, wb_ref,
              bb_ref, o_ref):
    eps = eps_ref[0, 0]
    z = (1.0 + eps) * h_ref[...] + agg_ref[0] + agg_ref[1]
    a = jnp.dot(z.astype(jnp.bfloat16), wa_ref[...].astype(jnp.bfloat16),
                preferred_element_type=jnp.float32)
    a = jnp.maximum(a + ba_ref[...], 0.0)
    y = jnp.dot(a.astype(jnp.bfloat16), wb_ref[...].astype(jnp.bfloat16),
                preferred_element_type=jnp.float32)
    y = y + bb_ref[...]
    if relu_out:
        y = jnp.maximum(y, 0.0)
    o_ref[...] = y


def _mlp(relu_out, eps, h, agg, wa, ba, wb, bb):
    hid = wa.shape[1]
    return pl.pallas_call(
        functools.partial(_mlp_body, relu_out),
        grid=(N // BR,),
        in_specs=[
            pl.BlockSpec(memory_space=pltpu.SMEM),
            pl.BlockSpec((BR, DIM), lambda i: (i, 0)),
            pl.BlockSpec((NC, BR, DIM), lambda i: (0, i, 0)),
            pl.BlockSpec((DIM, hid), lambda i: (0, 0)),
            pl.BlockSpec((1, hid), lambda i: (0, 0)),
            pl.BlockSpec((hid, DIM), lambda i: (0, 0)),
            pl.BlockSpec((1, DIM), lambda i: (0, 0)),
        ],
        out_specs=pl.BlockSpec((BR, DIM), lambda i: (i, 0)),
        out_shape=jax.ShapeDtypeStruct((N, DIM), jnp.float32),
    )(eps, h, agg, wa, ba, wb, bb)


def kernel(x, edge_index, W_emb,
           eps1, W1a, b1a, W1b, b1b,
           eps2, W2a, b2a, W2b, b2b,
           eps3, W3a, b3a, W3b, b3b):
    src3 = edge_index[0].reshape(NW * NG, GC, CH)
    dst3 = edge_index[1].reshape(NW * NG, GC, CH)
    zeros = jnp.zeros((N, DIM), jnp.float32)

    h = _embed(x, W_emb)
    layers = [
        (eps1, W1a, b1a, W1b, b1b, False),
        (eps2, W2a, b2a, W2b, b2b, True),
        (eps3, W3a, b3a, W3b, b3b, True),
    ]
    for eps, wa, ba, wb, bb, relu_out in layers:
        part = _segment_sum_sc(src3, dst3, h, zeros)
        agg = part.reshape(NC, N, DIM)
        h = _mlp(relu_out, jnp.reshape(eps, (1, 1)), h, agg,
                 wa, jnp.reshape(ba, (1, -1)), wb, jnp.reshape(bb, (1, -1)))
    return h


# speedup vs baseline: 12.5266x; 1.0260x over previous
import functools

import jax
import jax.numpy as jnp
from jax import lax
from jax.experimental import pallas as pl
from jax.experimental.pallas import tpu as pltpu
from jax.experimental.pallas import tpu_sc as plsc

N = 10000
E = 320000
DIM = 128

NC = 2
NS = 16
NW = NC * NS
EPW = E // NW
CH = 100
NCHUNK = EPW // CH
NB = 3
GC = 5
NG = NCHUNK // GC
RPT = 624
TAIL0 = NS * RPT
TAILN = N - TAIL0


def _seg_sum_body(src_hbm, dst_hbm, h_hbm, zero_hbm, out_hbm,
                  src_v, dst_v, rows_v, acc_sh, gsem, ssem, isem):
    c = lax.axis_index("c")
    s = lax.axis_index("s")
    wid = c * NS + s

    r0 = pl.multiple_of(s * RPT, 8)
    pltpu.sync_copy(zero_hbm.at[pl.ds(r0, RPT)], acc_sh.at[pl.ds(r0, RPT)])

    @pl.when(s == NS - 1)
    def _zero_tail():
        pltpu.sync_copy(zero_hbm.at[pl.ds(TAIL0, TAILN)],
                        acc_sh.at[pl.ds(TAIL0, TAILN)])

    def _idx_start(g, gb):
        pltpu.make_async_copy(src_hbm.at[wid * NG + g],
                              src_v.at[pl.ds(gb * GC, GC)], isem).start()
        pltpu.make_async_copy(dst_hbm.at[wid * NG + g],
                              dst_v.at[pl.ds(gb * GC, GC)], isem).start()

    def _idx_wait(g, gb):
        pltpu.make_async_copy(src_hbm.at[wid * NG + g],
                              src_v.at[pl.ds(gb * GC, GC)], isem).wait()
        pltpu.make_async_copy(dst_hbm.at[wid * NG + g],
                              dst_v.at[pl.ds(gb * GC, GC)], isem).wait()

    _idx_start(0, 0)
    plsc.subcore_barrier()

    def _gather(j, b):
        r = lax.rem(j, 2 * GC)
        return pltpu.make_async_copy(h_hbm.at[src_v.at[r]], rows_v.at[b], gsem)

    def _scatter(j, b):
        r = lax.rem(j, 2 * GC)
        return pltpu.async_copy(rows_v.at[b], acc_sh.at[dst_v.at[r]], ssem,
                                add=True)

    def _scatter_wait(j, b):
        r = lax.rem(j, 2 * GC)
        pltpu.make_async_copy(rows_v.at[b], acc_sh.at[dst_v.at[r]],
                              ssem).wait()

    _idx_wait(0, 0)

    @pl.when(NG > 1)
    def _prefetch_first():
        _idx_start(1, 1)

    _gather(0, 0).start()

    def step(j, carry):
        b = lax.rem(j, NB)

        @pl.when(j >= 2)
        def _drain_scatter():
            _scatter_wait(j - 2, lax.rem(j + 1, NB))

        @pl.when(j + 1 < NCHUNK)
        def _issue_gather():
            nj = j + 1
            g = nj // GC
            gb = lax.rem(g, 2)

            @pl.when(lax.rem(nj, GC) == 0)
            def _rotate_idx_group():
                _idx_wait(g, gb)

            @pl.when((lax.rem(nj, GC) == 3) & (nj >= GC) & (g + 1 < NG))
            def _prefetch_next():
                _idx_start(g + 1, 1 - gb)

            _gather(nj, lax.rem(nj, NB)).start()

        _gather(j, b).wait()
        _scatter(j, b)
        return carry

    lax.fori_loop(0, NCHUNK, step, 0)
    _scatter_wait(NCHUNK - 2, lax.rem(NCHUNK - 2, NB))
    _scatter_wait(NCHUNK - 1, lax.rem(NCHUNK - 1, NB))
    plsc.subcore_barrier()

    o0 = pl.multiple_of(c * N + r0, 8)
    pltpu.sync_copy(acc_sh.at[pl.ds(r0, RPT)], out_hbm.at[pl.ds(o0, RPT)])

    @pl.when(s == NS - 1)
    def _out_tail():
        ot = pl.multiple_of(c * N + TAIL0, 8)
        pltpu.sync_copy(acc_sh.at[pl.ds(TAIL0, TAILN)],
                        out_hbm.at[pl.ds(ot, TAILN)])


@functools.partial(
    pl.kernel,
    out_type=jax.ShapeDtypeStruct((NC * N, DIM), jnp.float32),
    mesh=plsc.VectorSubcoreMesh(core_axis_name="c", subcore_axis_name="s"),
    scratch_types=[
        pltpu.VMEM((2 * GC, CH), jnp.int32),
        pltpu.VMEM((2 * GC, CH), jnp.int32),
        pltpu.VMEM((NB, CH, DIM), jnp.float32),
        pltpu.VMEM_SHARED((N, DIM), jnp.float32),
        pltpu.SemaphoreType.DMA,
        pltpu.SemaphoreType.DMA,
        pltpu.SemaphoreType.DMA,
    ],
)
def _segment_sum_sc(src_hbm, dst_hbm, h_hbm, zero_hbm, out_hbm,
                    src_v, dst_v, rows_v, acc_sh, gsem, ssem, isem):
    _seg_sum_body(src_hbm, dst_hbm, h_hbm, zero_hbm, out_hbm,
                  src_v, dst_v, rows_v, acc_sh, gsem, ssem, isem)


BR = 2000


def _embed_body(x_ref, w_ref, o_ref):
    o_ref[...] = jnp.dot(x_ref[...].astype(jnp.bfloat16),
                         w_ref[...].astype(jnp.bfloat16),
                         preferred_element_type=jnp.float32)


def _embed(x, w):
    return pl.pallas_call(
        _embed_body,
        grid=(N // BR,),
        in_specs=[
            pl.BlockSpec((BR, DIM), lambda i: (i, 0)),
            pl.BlockSpec((DIM, DIM), lambda i: (0, 0)),
        ],
        out_specs=pl.BlockSpec((BR, DIM), lambda i: (i, 0)),
        out_shape=jax.ShapeDtypeStruct((N, DIM), jnp.float32),
    )(x, w)


def _mlp_body(relu_out, eps_ref, h_ref, agg_ref, wa_ref, ba_ref, wb_ref,
              bb_ref, o_ref):
    eps = eps_ref[0, 0]
    z = (1.0 + eps) * h_ref[...] + agg_ref[0] + agg_ref[1]
    a = jnp.dot(z.astype(jnp.bfloat16), wa_ref[...].astype(jnp.bfloat16),
                preferred_element_type=jnp.float32)
    a = jnp.maximum(a + ba_ref[...], 0.0)
    y = jnp.dot(a.astype(jnp.bfloat16), wb_ref[...].astype(jnp.bfloat16),
                preferred_element_type=jnp.float32)
    y = y + bb_ref[...]
    if relu_out:
        y = jnp.maximum(y, 0.0)
    o_ref[...] = y


def _mlp(relu_out, eps, h, agg, wa, ba, wb, bb):
    hid = wa.shape[1]
    return pl.pallas_call(
        functools.partial(_mlp_body, relu_out),
        grid=(N // BR,),
        in_specs=[
            pl.BlockSpec(memory_space=pltpu.SMEM),
            pl.BlockSpec((BR, DIM), lambda i: (i, 0)),
            pl.BlockSpec((NC, BR, DIM), lambda i: (0, i, 0)),
            pl.BlockSpec((DIM, hid), lambda i: (0, 0)),
            pl.BlockSpec((1, hid), lambda i: (0, 0)),
            pl.BlockSpec((hid, DIM), lambda i: (0, 0)),
            pl.BlockSpec((1, DIM), lambda i: (0, 0)),
        ],
        out_specs=pl.BlockSpec((BR, DIM), lambda i: (i, 0)),
        out_shape=jax.ShapeDtypeStruct((N, DIM), jnp.float32),
    )(eps, h, agg, wa, ba, wb, bb)


def kernel(x, edge_index, W_emb,
           eps1, W1a, b1a, W1b, b1b,
           eps2, W2a, b2a, W2b, b2b,
           eps3, W3a, b3a, W3b, b3b):
    src3 = edge_index[0].reshape(NW * NG, GC, CH)
    dst3 = edge_index[1].reshape(NW * NG, GC, CH)
    zeros = jnp.zeros((N, DIM), jnp.float32)

    h = _embed(x, W_emb)
    layers = [
        (eps1, W1a, b1a, W1b, b1b, False),
        (eps2, W2a, b2a, W2b, b2b, True),
        (eps3, W3a, b3a, W3b, b3b, True),
    ]
    for eps, wa, ba, wb, bb, relu_out in layers:
        part = _segment_sum_sc(src3, dst3, h, zeros)
        agg = part.reshape(NC, N, DIM)
        h = _mlp(relu_out, jnp.reshape(eps, (1, 1)), h, agg,
                 wa, jnp.reshape(ba, (1, -1)), wb, jnp.reshape(bb, (1, -1)))
    return h
